# Initial kernel scaffold; baseline (speedup 1.0000x reference)
import jax
import jax.numpy as jnp
from jax.experimental import pallas as pl


def kernel(x, W1, b1, W2, b2, W3, b3):
    raise NotImplementedError("write your pallas kernel here")


# speedup vs baseline: 26.3429x; 26.3429x over previous
import functools

import jax
import jax.numpy as jnp
from jax.experimental import pallas as pl

B, P, F = 1024, 128, 6
K = 3
C = 8


def _body(xT_ref, cC_ref, W1cT_ref, b1_ref, W2T_ref, b2_ref, W3T_ref, b3_ref,
          out_ref):
    xT = xT_ref[...]
    c0r = xT[:, 1:2, :]
    c1r = xT[:, 2:3, :]
    c0c = cC_ref[:, :, 0:1]
    c1c = cC_ref[:, :, 1:2]
    dx = c0c - c0r
    dy = c1c - c1r
    iota_s = jax.lax.broadcasted_iota(jnp.int32, (C, P, P), 1)
    iota_l = jax.lax.broadcasted_iota(jnp.int32, (C, P, P), 2)
    d2 = dx * dx + dy * dy
    d2 = d2 + jnp.where(iota_s == iota_l, jnp.float32(1e9), jnp.float32(0.0))

    U = jnp.einsum("kf,cfp->ckp", W1cT_ref[...], xT,
                   preferred_element_type=jnp.float32)
    A = U[:, :32, :] + b1_ref[...]
    Bm = U[:, 32:, :]

    acc = jnp.zeros((C, 32, P), dtype=jnp.float32)
    for _ in range(K):
        m = jnp.min(d2, axis=1, keepdims=True)
        idx = jnp.min(jnp.where(d2 == m, iota_s, P), axis=1,
                      keepdims=True)
        onehotT = (iota_s == idx)
        oh = onehotT.astype(jnp.float32)
        Gk = jnp.einsum("chj,cji->chi", Bm, oh,
                        preferred_element_type=jnp.float32)
        h1 = jnp.maximum(A + Gk, 0.0)
        h2 = jnp.maximum(
            jnp.einsum("hg,cgp->chp", W2T_ref[...], h1,
                       preferred_element_type=jnp.float32) + b2_ref[...], 0.0)
        acc = acc + h2
        d2 = jnp.where(onehotT, jnp.float32(2e9), d2)

    agg = acc * jnp.float32(1.0 / K)
    pooled = jnp.sum(agg, axis=2, keepdims=True) * jnp.float32(1.0 / P)
    logits = jnp.einsum("oh,chs->cos", W3T_ref[...], pooled,
                        preferred_element_type=jnp.float32) + b3_ref[...]
    z = logits - jnp.max(logits, axis=1, keepdims=True)
    e = jnp.exp(z)
    out_ref[...] = e / jnp.sum(e, axis=1, keepdims=True)


@jax.jit
def kernel(x, W1, b1, W2, b2, W3, b3):
    x8 = jnp.pad(x, ((0, 0), (0, 0), (0, 2)))
    xT = x8.transpose(0, 2, 1)
    cC = x[:, :, 1:3]
    W1c = jnp.concatenate([W1[:F] - W1[F:], W1[F:]], axis=1)
    W1cT = jnp.pad(W1c.T, ((0, 0), (0, 2)))
    b1c = b1.reshape(1, 32, 1)
    b2c = b2.reshape(1, 32, 1)
    b3c = b3.reshape(1, 2, 1)

    grid = (B // C,)
    out = pl.pallas_call(
        _body,
        grid=grid,
        in_specs=[
            pl.BlockSpec((C, 8, P), lambda i: (i, 0, 0)),
            pl.BlockSpec((C, P, 2), lambda i: (i, 0, 0)),
            pl.BlockSpec((64, 8), lambda i: (0, 0)),
            pl.BlockSpec((1, 32, 1), lambda i: (0, 0, 0)),
            pl.BlockSpec((32, 32), lambda i: (0, 0)),
            pl.BlockSpec((1, 32, 1), lambda i: (0, 0, 0)),
            pl.BlockSpec((2, 32), lambda i: (0, 0)),
            pl.BlockSpec((1, 2, 1), lambda i: (0, 0, 0)),
        ],
        out_specs=pl.BlockSpec((C, 2, 1), lambda i: (i, 0, 0)),
        out_shape=jax.ShapeDtypeStruct((B, 2, 1), jnp.float32),
    )(xT, cC, W1cT, b1c, W2.T, b2c, W3.T, b3c)
    return out.reshape(B, 2)
